# baseline (speedup 1.0000x reference)
import functools

import jax
import jax.numpy as jnp
from jax import lax
from jax.experimental import pallas as pl
from jax.experimental.pallas import tpu as pltpu
from jax.experimental.pallas import tpu_sc as plsc

_B = 16384
_L = 20
_EMBED = 64
_N = _B * _L
_NW = 32
_PER_W = _N // _NW
_CHUNK = 1280
_NCHUNK = _PER_W // _CHUNK
_GROUPS = _CHUNK // _L
_VPR = _EMBED // 16

_mesh = plsc.VectorSubcoreMesh(
    core_axis_name="c", subcore_axis_name="s", num_cores=2, num_subcores=16
)


@functools.partial(
    pl.kernel,
    out_type=jax.ShapeDtypeStruct((_N, _EMBED), jnp.float32),
    mesh=_mesh,
    scratch_types=[
        pltpu.VMEM((_CHUNK,), jnp.int32),
        pltpu.VMEM((_CHUNK, _EMBED), jnp.float32),
        pltpu.VMEM((32, _EMBED), jnp.float32),
        pltpu.SemaphoreType.DMA,
    ],
    compiler_params=pltpu.CompilerParams(use_tc_tiling_on_sc=False),
)
def _emb_lookup(x_hbm, wemb_hbm, pemb_hbm, out_hbm, idx_v, rows_v, pos_v, sem):
    wid = lax.axis_index("s") * 2 + lax.axis_index("c")
    pltpu.sync_copy(pemb_hbm, pos_v)

    def chunk_body(ci, carry):
        base = wid * _PER_W + ci * _CHUNK
        pltpu.sync_copy(x_hbm.at[pl.ds(base, _CHUNK)], idx_v)
        pltpu.async_copy(wemb_hbm.at[idx_v], rows_v, sem).wait()

        def add_body(g, c2):
            r0 = g * _L
            for p in range(_L):
                for d in range(_VPR):
                    sl = pl.ds(d * 16, 16)
                    rows_v[r0 + p, sl] = rows_v[r0 + p, sl] + pos_v[p, sl]
            return c2

        lax.fori_loop(0, _GROUPS, add_body, 0)
        pltpu.sync_copy(rows_v, out_hbm.at[pl.ds(base, _CHUNK)])
        return carry

    lax.fori_loop(0, _NCHUNK, chunk_body, 0)


def kernel(x, word_emb, pos_emb):
    out = _emb_lookup(x.reshape(_N), word_emb, pos_emb)
    return out.reshape(_B, _L, _EMBED)
